# baseline (speedup 1.0000x reference)
import functools

import jax
import jax.numpy as jnp
from jax import lax
from jax.experimental import pallas as pl
from jax.experimental.pallas import tpu as pltpu
from jax.experimental.pallas import tpu_sc as plsc

NUM_EXPERTS = 64
TOP_K = 8
B = 32768

_NC = 2
_NS = 16
_NW = _NC * _NS
_TOK_W = B // _NW
_CHUNK = _TOK_W // 2


@functools.partial(
    pl.kernel,
    out_type=(
        jax.ShapeDtypeStruct((TOP_K, B), jnp.float32),
        jax.ShapeDtypeStruct((TOP_K, B), jnp.int32),
    ),
    mesh=plsc.VectorSubcoreMesh(core_axis_name="c", subcore_axis_name="s"),
    compiler_params=pltpu.CompilerParams(needs_layout_passes=False),
    scratch_types=[
        pltpu.VMEM((_CHUNK, NUM_EXPERTS), jnp.float32),
        pltpu.VMEM((TOP_K, _TOK_W), jnp.float32),
        pltpu.VMEM((TOP_K, _TOK_W), jnp.int32),
    ],
)
def _router(logits_hbm, vals_hbm, idx_hbm, lbuf, vbuf, ibuf):
    wid = lax.axis_index("s") * _NC + lax.axis_index("c")
    base = wid * _TOK_W

    lane = lax.iota(jnp.int32, 16)
    mask8 = lane < 8
    row8 = lane & 7
    rot8 = lane ^ 8
    rot4 = lane ^ 4
    rot2 = lane ^ 2
    rot1 = lane ^ 1

    def allsum(x):
        x = x + x[rot8]
        x = x + x[rot4]
        x = x + x[rot2]
        return x + x[rot1]

    def half_clean(ka, va, kb, vb):
        ck = jnp.where(mask8, ka, kb)
        cv = jnp.where(mask8, va, vb)
        rk = ck[rot8]
        rv = cv[rot8]
        ge = ck >= rk
        return jnp.where(ge, ck, rk), jnp.where(ge, cv, rv)

    for chunk in range(2):
        pltpu.sync_copy(
            logits_hbm.at[pl.ds(base + chunk * _CHUNK, _CHUNK)], lbuf)

        @plsc.parallel_loop(0, _CHUNK, unroll=4)
        def _token(t, chunk=chunk):
            e0 = jnp.exp(lbuf[t, pl.ds(0, 16)])
            e1 = jnp.exp(lbuf[t, pl.ds(16, 16)])
            e2 = jnp.exp(lbuf[t, pl.ds(32, 16)])
            e3 = jnp.exp(lbuf[t, pl.ds(48, 16)])
            z = allsum((e0 + e1) + (e2 + e3))

            k0, v0 = plsc.sort_key_val(e0, lane, descending=True)
            k1, v1 = plsc.sort_key_val(e1, lane + 16, descending=False)
            k2, v2 = plsc.sort_key_val(e2, lane + 32, descending=True)
            k3, v3 = plsc.sort_key_val(e3, lane + 48, descending=False)

            d01k, d01v = half_clean(k0, v0, k1, v1)
            d23k, d23v = half_clean(k2, v2, k3, v3)

            fk, fv = plsc.sort_key_val(
                jnp.where(mask8, d01k, d23k), jnp.where(mask8, d01v, d23v),
                descending=True)

            s8 = allsum(jnp.where(mask8, fk, 0.0))
            r = 1.0 / (s8 + 1e-6 * z)

            col = jnp.full((16,), chunk * _CHUNK + t, jnp.int32)
            plsc.store_scatter(vbuf, [row8, col], fk * r, mask=mask8)
            plsc.store_scatter(ibuf, [row8, col], fv, mask=mask8)

    pltpu.sync_copy(vbuf, vals_hbm.at[:, pl.ds(base, _TOK_W)])
    pltpu.sync_copy(ibuf, idx_hbm.at[:, pl.ds(base, _TOK_W)])


def kernel(logits, noise_std, training):
    del noise_std, training
    vals, idx = _router(logits)
    return vals.T, idx.T


# speedup vs baseline: 1.0364x; 1.0364x over previous
import functools

import jax
import jax.numpy as jnp
from jax import lax
from jax.experimental import pallas as pl
from jax.experimental.pallas import tpu as pltpu
from jax.experimental.pallas import tpu_sc as plsc

NUM_EXPERTS = 64
TOP_K = 8
B = 32768

_NC = 2
_NS = 16
_NW = _NC * _NS
_TOK_W = B // _NW
_CHUNK = _TOK_W // 2


@functools.partial(
    pl.kernel,
    out_type=(
        jax.ShapeDtypeStruct((TOP_K, B), jnp.float32),
        jax.ShapeDtypeStruct((TOP_K, B), jnp.int32),
    ),
    mesh=plsc.VectorSubcoreMesh(core_axis_name="c", subcore_axis_name="s"),
    compiler_params=pltpu.CompilerParams(
        needs_layout_passes=False,
        skip_device_barrier=True,
        disable_bounds_checks=True,
        allow_input_fusion=[True],
    ),
    scratch_types=[
        pltpu.VMEM((_CHUNK, NUM_EXPERTS), jnp.float32),
        pltpu.VMEM((TOP_K, _TOK_W), jnp.float32),
        pltpu.VMEM((TOP_K, _TOK_W), jnp.int32),
    ],
)
def _router(logits_hbm, vals_hbm, idx_hbm, lbuf, vbuf, ibuf):
    wid = lax.axis_index("s") * _NC + lax.axis_index("c")
    base = wid * _TOK_W

    lane = lax.iota(jnp.int32, 16)
    mask8 = lane < 8
    row8 = lane & 7
    rot8 = lane ^ 8

    def half_clean(ka, va, kb, vb):
        ck = jnp.where(mask8, ka, kb)
        cv = jnp.where(mask8, va, vb)
        rk = ck[rot8]
        rv = cv[rot8]
        ge = ck >= rk
        return jnp.where(ge, ck, rk), jnp.where(ge, cv, rv)

    for chunk in range(2):
        pltpu.sync_copy(
            logits_hbm.at[pl.ds(base + chunk * _CHUNK, _CHUNK)], lbuf)

        @plsc.parallel_loop(0, _CHUNK, unroll=4)
        def _token(t, chunk=chunk):
            e0 = jnp.exp(lbuf[t, pl.ds(0, 16)])
            e1 = jnp.exp(lbuf[t, pl.ds(16, 16)])
            e2 = jnp.exp(lbuf[t, pl.ds(32, 16)])
            e3 = jnp.exp(lbuf[t, pl.ds(48, 16)])
            zc = plsc.cumsum((e0 + e1) + (e2 + e3))
            z = zc[jnp.full((16,), 15, jnp.int32)]

            k0, v0 = plsc.sort_key_val(e0, lane, descending=True)
            k1, v1 = plsc.sort_key_val(e1, lane + 16, descending=False)
            k2, v2 = plsc.sort_key_val(e2, lane + 32, descending=True)
            k3, v3 = plsc.sort_key_val(e3, lane + 48, descending=False)

            d01k, d01v = half_clean(k0, v0, k1, v1)
            d23k, d23v = half_clean(k2, v2, k3, v3)

            fk, fv = plsc.sort_key_val(
                jnp.where(mask8, d01k, d23k), jnp.where(mask8, d01v, d23v),
                descending=True)

            s8 = plsc.cumsum(fk)[jnp.full((16,), TOP_K - 1, jnp.int32)]
            r = 1.0 / (s8 + 1e-6 * z)

            col = jnp.full((16,), chunk * _CHUNK + t, jnp.int32)
            plsc.store_scatter(vbuf, [row8, col], fk * r, mask=mask8)
            plsc.store_scatter(ibuf, [row8, col], fv, mask=mask8)

    pltpu.sync_copy(vbuf, vals_hbm.at[:, pl.ds(base, _TOK_W)])
    pltpu.sync_copy(ibuf, idx_hbm.at[:, pl.ds(base, _TOK_W)])


def kernel(logits, noise_std, training):
    del noise_std, training
    vals, idx = _router(logits)
    return vals.T, idx.T


# speedup vs baseline: 1.0551x; 1.0181x over previous
import functools

import jax
import jax.numpy as jnp
from jax import lax
from jax.experimental import pallas as pl
from jax.experimental.pallas import tpu as pltpu
from jax.experimental.pallas import tpu_sc as plsc

NUM_EXPERTS = 64
TOP_K = 8
B = 32768

_NC = 2
_NS = 16
_NW = _NC * _NS
_TOK_W = B // _NW
_NCHUNK = 4
_CHUNK = _TOK_W // _NCHUNK


@functools.partial(
    pl.kernel,
    out_type=(
        jax.ShapeDtypeStruct((TOP_K, B), jnp.float32),
        jax.ShapeDtypeStruct((TOP_K, B), jnp.int32),
    ),
    mesh=plsc.VectorSubcoreMesh(core_axis_name="c", subcore_axis_name="s"),
    compiler_params=pltpu.CompilerParams(needs_layout_passes=False),
    scratch_types=[
        pltpu.VMEM((_CHUNK, NUM_EXPERTS), jnp.float32),
        pltpu.VMEM((_CHUNK, NUM_EXPERTS), jnp.float32),
        pltpu.VMEM((TOP_K, _TOK_W), jnp.float32),
        pltpu.VMEM((TOP_K, _TOK_W), jnp.int32),
        pltpu.SemaphoreType.DMA,
        pltpu.SemaphoreType.DMA,
    ],
)
def _router(logits_hbm, vals_hbm, idx_hbm, lbufa, lbufb, vbuf, ibuf, sema, semb):
    wid = lax.axis_index("s") * _NC + lax.axis_index("c")
    base = wid * _TOK_W
    bufs = (lbufa, lbufb)
    sems = (sema, semb)

    lane = lax.iota(jnp.int32, 16)
    mask8 = lane < 8
    row8 = lane & 7
    rot8 = lane ^ 8

    def half_clean(ka, va, kb, vb):
        ck = jnp.where(mask8, ka, kb)
        cv = jnp.where(mask8, va, vb)
        rk = ck[rot8]
        rv = cv[rot8]
        ge = ck >= rk
        return jnp.where(ge, ck, rk), jnp.where(ge, cv, rv)

    def start(chunk):
        return pltpu.async_copy(
            logits_hbm.at[pl.ds(base + chunk * _CHUNK, _CHUNK)],
            bufs[chunk % 2], sems[chunk % 2])

    pending = start(0)
    for chunk in range(_NCHUNK):
        pending.wait()
        if chunk + 1 < _NCHUNK:
            pending = start(chunk + 1)
        lbuf = bufs[chunk % 2]

        @plsc.parallel_loop(0, _CHUNK, unroll=8)
        def _token(t, chunk=chunk, lbuf=lbuf):
            e0 = jnp.exp(lbuf[t, pl.ds(0, 16)])
            e1 = jnp.exp(lbuf[t, pl.ds(16, 16)])
            e2 = jnp.exp(lbuf[t, pl.ds(32, 16)])
            e3 = jnp.exp(lbuf[t, pl.ds(48, 16)])
            zc = plsc.cumsum((e0 + e1) + (e2 + e3))
            z = zc[jnp.full((16,), 15, jnp.int32)]

            k0, v0 = plsc.sort_key_val(e0, lane, descending=True)
            k1, v1 = plsc.sort_key_val(e1, lane + 16, descending=False)
            k2, v2 = plsc.sort_key_val(e2, lane + 32, descending=True)
            k3, v3 = plsc.sort_key_val(e3, lane + 48, descending=False)

            d01k, d01v = half_clean(k0, v0, k1, v1)
            d23k, d23v = half_clean(k2, v2, k3, v3)

            fk, fv = plsc.sort_key_val(
                jnp.where(mask8, d01k, d23k), jnp.where(mask8, d01v, d23v),
                descending=True)

            s8 = plsc.cumsum(fk)[jnp.full((16,), TOP_K - 1, jnp.int32)]
            r = 1.0 / (s8 + 1e-6 * z)

            col = jnp.full((16,), chunk * _CHUNK + t, jnp.int32)
            plsc.store_scatter(vbuf, [row8, col], fk * r, mask=mask8)
            plsc.store_scatter(ibuf, [row8, col], fv, mask=mask8)

    pltpu.sync_copy(vbuf, vals_hbm.at[:, pl.ds(base, _TOK_W)])
    pltpu.sync_copy(ibuf, idx_hbm.at[:, pl.ds(base, _TOK_W)])


def kernel(logits, noise_std, training):
    del noise_std, training
    vals, idx = _router(logits)
    return vals.T, idx.T
